# speedup vs baseline: 2.1815x; 2.1815x over previous
import functools

import jax
import jax.numpy as jnp
from jax import lax
from jax.experimental import pallas as pl
from jax.experimental.pallas import tpu as pltpu
from jax.experimental.pallas import tpu_sc as plsc

_NC = 2
_NS = 16
_NW = _NC * _NS


@functools.partial(jax.jit, static_argnames=("chunk",))
def _gather_rows(position_ids, table, chunk=64):
    (bsz, seq) = position_ids.shape
    (vocab, dim) = table.shape
    total = bsz * seq
    b_per_w = total // _NW
    n_chunks = b_per_w // chunk

    idx2d = position_ids.reshape(_NW * n_chunks, chunk)

    mesh = plsc.VectorSubcoreMesh(core_axis_name="c", subcore_axis_name="s")

    @functools.partial(
        pl.kernel,
        mesh=mesh,
        out_type=jax.ShapeDtypeStruct((total, dim), jnp.float32),
        scratch_types=[
            pltpu.VMEM((n_chunks, chunk), jnp.int32),
            pltpu.VMEM((chunk, dim), jnp.float32),
            pltpu.SemaphoreType.DMA,
        ],
    )
    def k(idx_hbm, table_hbm, out_hbm, idx_v, rows_v, sem):
        wid = lax.axis_index("s") * _NC + lax.axis_index("c")
        base = wid * b_per_w
        pltpu.sync_copy(idx_hbm.at[pl.ds(wid * n_chunks, n_chunks)], idx_v)

        def body(j, _):
            pltpu.async_copy(table_hbm.at[idx_v.at[j]], rows_v, sem).wait()
            pltpu.sync_copy(
                rows_v, out_hbm.at[pl.ds(base + j * chunk, chunk)]
            )
            return 0

        lax.fori_loop(0, n_chunks, body, 0)

    out = k(idx2d, table)
    return out.reshape(bsz, seq, dim)


def kernel(position_ids, table):
    return _gather_rows(position_ids.astype(jnp.int32), table)


# speedup vs baseline: 2.3638x; 1.0836x over previous
import functools

import jax
import jax.numpy as jnp
from jax import lax
from jax.experimental import pallas as pl
from jax.experimental.pallas import tpu as pltpu
from jax.experimental.pallas import tpu_sc as plsc

_NC = 2
_NS = 16
_NW = _NC * _NS


@functools.partial(jax.jit, static_argnames=("chunk",))
def _gather_rows(position_ids, table, chunk=32):
    (bsz, seq) = position_ids.shape
    (vocab, dim) = table.shape
    total = bsz * seq
    b_per_w = total // _NW
    n_chunks = b_per_w // chunk
    assert n_chunks % 2 == 0

    idx2d = position_ids.reshape(_NW * n_chunks, chunk)

    mesh = plsc.VectorSubcoreMesh(core_axis_name="c", subcore_axis_name="s")

    @functools.partial(
        pl.kernel,
        mesh=mesh,
        out_type=jax.ShapeDtypeStruct((total, dim), jnp.float32),
        scratch_types=[
            pltpu.VMEM((n_chunks, chunk), jnp.int32),
            pltpu.VMEM((chunk, dim), jnp.float32),
            pltpu.VMEM((chunk, dim), jnp.float32),
            pltpu.SemaphoreType.DMA,
            pltpu.SemaphoreType.DMA,
        ],
    )
    def k(idx_hbm, table_hbm, out_hbm, idx_v, rows0, rows1, sem0, sem1):
        wid = lax.axis_index("s") * _NC + lax.axis_index("c")
        base = wid * b_per_w
        pltpu.sync_copy(idx_hbm.at[pl.ds(wid * n_chunks, n_chunks)], idx_v)

        def start(j, rows, sem):
            pltpu.async_copy(table_hbm.at[idx_v.at[j]], rows, sem)

        def wait(rows, sem):
            pltpu.make_async_copy(table_hbm.at[idx_v.at[0]], rows, sem).wait()

        def write(j, rows):
            pltpu.sync_copy(rows, out_hbm.at[pl.ds(base + j * chunk, chunk)])

        start(0, rows0, sem0)

        def body(i, _):
            j0 = 2 * i
            start(j0 + 1, rows1, sem1)
            wait(rows0, sem0)
            write(j0, rows0)

            @pl.when(j0 + 2 < n_chunks)
            def _():
                start(j0 + 2, rows0, sem0)

            wait(rows1, sem1)
            write(j0 + 1, rows1)
            return 0

        lax.fori_loop(0, n_chunks // 2, body, 0)

    out = k(idx2d, table)
    return out.reshape(bsz, seq, dim)


def kernel(position_ids, table):
    return _gather_rows(position_ids.astype(jnp.int32), table)


# speedup vs baseline: 2.3689x; 1.0022x over previous
import functools

import jax
import jax.numpy as jnp
from jax import lax
from jax.experimental import pallas as pl
from jax.experimental.pallas import tpu as pltpu
from jax.experimental.pallas import tpu_sc as plsc

_NC = 2
_NS = 16
_NW = _NC * _NS
_NBUF = 4


@functools.partial(jax.jit, static_argnames=("chunk",))
def _gather_rows(position_ids, table, chunk=16):
    (bsz, seq) = position_ids.shape
    (vocab, dim) = table.shape
    total = bsz * seq
    b_per_w = total // _NW
    n_chunks = b_per_w // chunk
    assert n_chunks % _NBUF == 0 and n_chunks >= 2 * _NBUF

    idx2d = position_ids.reshape(_NW * n_chunks, chunk)

    mesh = plsc.VectorSubcoreMesh(core_axis_name="c", subcore_axis_name="s")

    rows_t = pltpu.VMEM((chunk, dim), jnp.float32)

    @functools.partial(
        pl.kernel,
        mesh=mesh,
        out_type=jax.ShapeDtypeStruct((total, dim), jnp.float32),
        scratch_types=[
            pltpu.VMEM((n_chunks, chunk), jnp.int32),
            [rows_t] * _NBUF,
            [pltpu.SemaphoreType.DMA] * _NBUF,
            [pltpu.SemaphoreType.DMA] * _NBUF,
        ],
    )
    def k(idx_hbm, table_hbm, out_hbm, idx_v, rows, gsem, wsem):
        wid = lax.axis_index("s") * _NC + lax.axis_index("c")
        base = wid * b_per_w
        pltpu.sync_copy(idx_hbm.at[pl.ds(wid * n_chunks, n_chunks)], idx_v)

        def gstart(j, b):
            pltpu.async_copy(table_hbm.at[idx_v.at[j]], rows[b], gsem[b])

        def gwait(b):
            pltpu.make_async_copy(
                table_hbm.at[idx_v.at[0]], rows[b], gsem[b]
            ).wait()

        def wstart(j, b):
            pltpu.async_copy(
                rows[b], out_hbm.at[pl.ds(base + j * chunk, chunk)], wsem[b]
            )

        def wwait(b):
            pltpu.make_async_copy(
                rows[b], out_hbm.at[pl.ds(base, chunk)], wsem[b]
            ).wait()

        gstart(0, 0)
        gstart(1, 1)

        def body(i, _):
            for b in range(_NBUF):
                j = i * _NBUF + b
                gwait(b)
                wstart(j, b)
                bn = (b + 2) % _NBUF

                @pl.when(jnp.logical_and(j + 2 < n_chunks, j >= 2))
                def _():
                    wwait(bn)

                @pl.when(j + 2 < n_chunks)
                def _():
                    gstart(j + 2, bn)

            return 0

        lax.fori_loop(0, n_chunks // _NBUF, body, 0)

        for b in range(_NBUF):
            wwait(b)

    out = k(idx2d, table)
    return out.reshape(bsz, seq, dim)


def kernel(position_ids, table):
    return _gather_rows(position_ids.astype(jnp.int32), table)
